# Initial kernel scaffold; baseline (speedup 1.0000x reference)
import jax
import jax.numpy as jnp
from jax.experimental import pallas as pl


def kernel(x, edge_index, pos_edge_index, neg_edge_index, Wl1, Wr1, att1, bc1, Wl2, Wr2, att2, bc2, W1, b1, W2, b2, W3, b3, W4, b4):
    raise NotImplementedError("write your pallas kernel here")


# speedup vs baseline: 5.1677x; 5.1677x over previous
import jax
import jax.numpy as jnp
from jax import lax
from jax.experimental import pallas as pl
from jax.experimental.pallas import tpu as pltpu
from jax.experimental.pallas import tpu_sc as plsc

N = 10000
E = 320000
EP2 = 65536
NC, NS, LANES = 2, 16, 16
NW = NC * NS
NPS = 10240
CHK = NPS // NS
DH, DW = 16, 1024

_MESH = plsc.VectorSubcoreMesh(core_axis_name="c", subcore_axis_name="s")
_F32 = jnp.float32
_I32 = jnp.int32


def _m1_body(x_ref, wl_ref, wr_ref, xla, xlb, xra, xrb):
    xw = jnp.dot(x_ref[...], wl_ref[...], preferred_element_type=_F32)
    xv = jnp.dot(x_ref[...], wr_ref[...], preferred_element_type=_F32)
    xla[...] = xw[:, :128]
    xlb[...] = xw[:, 128:]
    xra[...] = xv[:, :128]
    xrb[...] = xv[:, 128:]


def _m1(x, Wl1, Wr1):
    BM = 1000
    return pl.pallas_call(
        _m1_body,
        grid=(N // BM,),
        in_specs=[pl.BlockSpec((BM, 128), lambda m: (m, 0)),
                  pl.BlockSpec((128, 256), lambda m: (0, 0)),
                  pl.BlockSpec((128, 256), lambda m: (0, 0))],
        out_specs=[pl.BlockSpec((BM, 128), lambda m: (m, 0))] * 4,
        out_shape=[jax.ShapeDtypeStruct((N, 128), _F32)] * 4,
    )(x, Wl1, Wr1)


def _m2_body(za_ref, zb_ref, wl_ref, wr_ref, xl2, xr2):
    za = za_ref[...]
    zb = zb_ref[...]
    xl2[...] = (jnp.dot(za, wl_ref[...][:128, :], preferred_element_type=_F32)
                + jnp.dot(zb, wl_ref[...][128:, :], preferred_element_type=_F32))
    xr2[...] = (jnp.dot(za, wr_ref[...][:128, :], preferred_element_type=_F32)
                + jnp.dot(zb, wr_ref[...][128:, :], preferred_element_type=_F32))


def _m2(z1a, z1b, Wl2, Wr2):
    BM = 1000
    return pl.pallas_call(
        _m2_body,
        grid=(N // BM,),
        in_specs=[pl.BlockSpec((BM, 128), lambda m: (m, 0)),
                  pl.BlockSpec((BM, 128), lambda m: (m, 0)),
                  pl.BlockSpec((256, 128), lambda m: (0, 0)),
                  pl.BlockSpec((256, 128), lambda m: (0, 0))],
        out_specs=[pl.BlockSpec((BM, 128), lambda m: (m, 0))] * 2,
        out_shape=[jax.ShapeDtypeStruct((N, 128), _F32)] * 2,
    )(z1a, z1b, Wl2, Wr2)


def _m3_body(p0_ref, p1_ref, bc_ref, w1_ref, zu, zv):
    z2 = p0_ref[...] + p1_ref[...] + bc_ref[...][None, :]
    zu[...] = jnp.dot(z2, w1_ref[...][:128, :], preferred_element_type=_F32)
    zv[...] = jnp.dot(z2, w1_ref[...][128:, :], preferred_element_type=_F32)


def _m3(p0, p1, bc2, W1):
    BM = 1000
    return pl.pallas_call(
        _m3_body,
        grid=(N // BM,),
        in_specs=[pl.BlockSpec((BM, 128), lambda m: (m, 0)),
                  pl.BlockSpec((BM, 128), lambda m: (m, 0)),
                  pl.BlockSpec((128,), lambda m: (0,)),
                  pl.BlockSpec((256, 128), lambda m: (0, 0))],
        out_specs=[pl.BlockSpec((BM, 128), lambda m: (m, 0))] * 2,
        out_shape=[jax.ShapeDtypeStruct((N, 128), _F32)] * 2,
    )(p0, p1, bc2, W1)


def _m4_body(h_ref, w2_ref, b2_ref, w3_ref, b3_ref, w4_ref, b4_ref, out):
    h2 = jnp.maximum(jnp.dot(h_ref[...], w2_ref[...],
                             preferred_element_type=_F32) + b2_ref[...][None, :], 0.0)
    h3 = jnp.maximum(jnp.dot(h2, w3_ref[...],
                             preferred_element_type=_F32) + b3_ref[...][None, :], 0.0)
    out[...] = jnp.dot(h3, w4_ref[...], preferred_element_type=_F32) + b4_ref[...]


def _m4(h1, W2, b2, W3, b3, W4, b4):
    BM = 2048
    o = pl.pallas_call(
        _m4_body,
        grid=(EP2 // BM,),
        in_specs=[pl.BlockSpec((BM, 128), lambda m: (m, 0)),
                  pl.BlockSpec((128, 128), lambda m: (0, 0)),
                  pl.BlockSpec((128,), lambda m: (0,)),
                  pl.BlockSpec((128, 64), lambda m: (0, 0)),
                  pl.BlockSpec((64,), lambda m: (0,)),
                  pl.BlockSpec((64, 1), lambda m: (0, 0)),
                  pl.BlockSpec((1,), lambda m: (0,))],
        out_specs=pl.BlockSpec((BM, 1), lambda m: (m, 0)),
        out_shape=jax.ShapeDtypeStruct((EP2, 1), _F32),
    )(h1, W2, b2, W3, b3, W4, b4)
    return jnp.squeeze(o, axis=1)


def _edge_logits_groups(pairs, attv, outv, maxv, tbuf, B):
    lane17 = lax.iota(_I32, LANES) * 17

    def group(g, _):
        for j in range(LANES):
            i = g * LANES + j
            acc = jnp.zeros((LANES,), _F32)
            for (bl, br, off) in pairs:
                for fc in range(128 // LANES):
                    sl = pl.ds(fc * LANES, LANES)
                    u = bl[i, sl] + br[i, sl]
                    t = jnp.maximum(u, 0.2 * u)
                    acc = acc + t * attv[pl.ds(off + fc * LANES, LANES)]
            tbuf[pl.ds(j * 17, LANES)] = acc
        gacc = plsc.load_gather(tbuf, [lane17])
        for k in range(1, LANES):
            gacc = gacc + plsc.load_gather(tbuf, [lane17 + k])
        outv[pl.ds(g * LANES, LANES)] = gacc
        maxv[...] = jnp.maximum(maxv[...], gacc)
        return 0

    lax.fori_loop(0, B // LANES, group, 0)


def _s1_split_body(xla, xlb, xra, xrb, atth, srch, dsth, lgh, wmh,
                   srcv, dstv, bla, blb, bra, brb, attv, outv, maxv, tbuf,
                   sem):
    B = 80
    EW = E // NW
    wid = lax.axis_index("s") * NC + lax.axis_index("c")
    base = wid * EW
    pltpu.sync_copy(atth, attv)
    maxv[...] = jnp.full((LANES,), -1e30, _F32)

    def batch(b, carry):
        off = base + b * B
        pltpu.sync_copy(srch.at[pl.ds(off, B)], srcv)
        pltpu.sync_copy(dsth.at[pl.ds(off, B)], dstv)
        d1 = pltpu.async_copy(xla.at[srcv], bla, sem)
        d2 = pltpu.async_copy(xlb.at[srcv], blb, sem)
        d3 = pltpu.async_copy(xra.at[dstv], bra, sem)
        d4 = pltpu.async_copy(xrb.at[dstv], brb, sem)
        d1.wait(); d2.wait(); d3.wait(); d4.wait()
        _edge_logits_groups([(bla, bra, 0), (blb, brb, 128)], attv, outv,
                            maxv, tbuf, B)
        pltpu.sync_copy(outv, lgh.at[pl.ds(off, B)])
        return carry

    lax.fori_loop(0, EW // B, batch, 0)
    pltpu.sync_copy(maxv, wmh.at[wid])


def _s1_split(xla, xlb, xra, xrb, att1, src, dst):
    B = 80
    return pl.kernel(
        _s1_split_body,
        out_type=[jax.ShapeDtypeStruct((E,), _F32),
                  jax.ShapeDtypeStruct((NW, LANES), _F32)],
        mesh=_MESH,
        compiler_params=pltpu.CompilerParams(needs_layout_passes=False),
        scratch_types=[pltpu.VMEM((B,), _I32), pltpu.VMEM((B,), _I32),
                       pltpu.VMEM((B, 128), _F32), pltpu.VMEM((B, 128), _F32),
                       pltpu.VMEM((B, 128), _F32), pltpu.VMEM((B, 128), _F32),
                       pltpu.VMEM((256,), _F32), pltpu.VMEM((B,), _F32),
                       pltpu.VMEM((LANES,), _F32),
                       pltpu.VMEM((17 * LANES,), _F32),
                       pltpu.SemaphoreType.DMA],
    )(xla, xlb, xra, xrb, att1, src, dst)


def _s1_full_body(xlh, xrh, atth, srch, dsth, lgh, wmh,
                  srcv, dstv, bl, br, attv, outv, maxv, tbuf, sem):
    B = 80
    EW = E // NW
    wid = lax.axis_index("s") * NC + lax.axis_index("c")
    base = wid * EW
    pltpu.sync_copy(atth, attv)
    maxv[...] = jnp.full((LANES,), -1e30, _F32)

    def batch(b, carry):
        off = base + b * B
        pltpu.sync_copy(srch.at[pl.ds(off, B)], srcv)
        pltpu.sync_copy(dsth.at[pl.ds(off, B)], dstv)
        d1 = pltpu.async_copy(xlh.at[srcv], bl, sem)
        d2 = pltpu.async_copy(xrh.at[dstv], br, sem)
        d1.wait(); d2.wait()
        _edge_logits_groups([(bl, br, 0)], attv, outv, maxv, tbuf, B)
        pltpu.sync_copy(outv, lgh.at[pl.ds(off, B)])
        return carry

    lax.fori_loop(0, EW // B, batch, 0)
    pltpu.sync_copy(maxv, wmh.at[wid])


def _s1_full(xl2, xr2, att2, src, dst):
    B = 80
    return pl.kernel(
        _s1_full_body,
        out_type=[jax.ShapeDtypeStruct((E,), _F32),
                  jax.ShapeDtypeStruct((NW, LANES), _F32)],
        mesh=_MESH,
        compiler_params=pltpu.CompilerParams(needs_layout_passes=False),
        scratch_types=[pltpu.VMEM((B,), _I32), pltpu.VMEM((B,), _I32),
                       pltpu.VMEM((B, 128), _F32), pltpu.VMEM((B, 128), _F32),
                       pltpu.VMEM((128,), _F32), pltpu.VMEM((B,), _F32),
                       pltpu.VMEM((LANES,), _F32),
                       pltpu.VMEM((17 * LANES,), _F32),
                       pltpu.SemaphoreType.DMA],
    )(xl2, xr2, att2, src, dst)


def _gmax(wmv):
    m = wmv[0]

    def mx(k, mm):
        return jnp.maximum(mm, wmv[k])

    m = lax.fori_loop(1, NW, mx, m)
    g = m[0]
    for k in range(1, LANES):
        g = jnp.maximum(g, m[k])
    return g


def _s2_body(lgh, dsth, wmh, dph, lgv, dstv, wmv, denl, sbuf, resv, shared):
    B2 = 2000
    EW = E // NW
    c = lax.axis_index("c")
    s = lax.axis_index("s")
    wid = s * NC + c
    base = wid * EW

    def zz(k, _):
        denl[pl.ds(k * LANES, LANES)] = jnp.zeros((LANES,), _F32)
        return 0

    lax.fori_loop(0, (DH * DW) // LANES, zz, 0)

    pltpu.sync_copy(wmh, wmv)
    g = _gmax(wmv)

    def batch(b, _):
        off = base + b * B2
        pltpu.sync_copy(lgh.at[pl.ds(off, B2)], lgv)
        pltpu.sync_copy(dsth.at[pl.ds(off, B2)], dstv)

        def grp(k, _2):
            sl = pl.ds(k * LANES, LANES)
            ex = jnp.exp(lgv[sl] - g)
            d = dstv[sl]
            plsc.addupdate_scatter(denl, [d], ex)
            return 0

        lax.fori_loop(0, B2 // LANES, grp, 0)
        return 0

    lax.fori_loop(0, EW // B2, batch, 0)

    for t in range(NS):
        pltpu.sync_copy(denl.at[pl.ds(t * DW, DW)], shared.at[t, s])
    plsc.subcore_barrier()
    pltpu.sync_copy(shared.at[s], sbuf)

    def col(k, _):
        sl = pl.ds(k * LANES, LANES)
        a = sbuf[0, sl]
        for p in range(1, NS):
            a = a + sbuf[p, sl]
        resv[sl] = a
        return 0

    lax.fori_loop(0, DW // LANES, col, 0)
    pltpu.sync_copy(resv, dph.at[c, pl.ds(s * DW, DW)])


def _s2(lg, dst, wm):
    return pl.kernel(
        _s2_body,
        out_type=jax.ShapeDtypeStruct((NC, DH * DW), _F32),
        mesh=_MESH,
        compiler_params=pltpu.CompilerParams(needs_layout_passes=False),
        scratch_types=[pltpu.VMEM((2000,), _F32), pltpu.VMEM((2000,), _I32),
                       pltpu.VMEM((NW, LANES), _F32),
                       pltpu.VMEM((DH * DW,), _F32),
                       pltpu.VMEM((NS, DW), _F32), pltpu.VMEM((DW,), _F32),
                       pltpu.VMEM_SHARED((NS, NS, DW), _F32)],
    )(lg, dst, wm)


def _denom_combine(dph, denv, dtmp):
    pltpu.sync_copy(dph.at[0], denv)
    for t in range(DH):
        pltpu.sync_copy(dph.at[1, pl.ds(t * DW, DW)], dtmp)

        def addp(k, _):
            sl = pl.ds(t * DW + k * LANES, LANES)
            denv[sl] = denv[sl] + dtmp[pl.ds(k * LANES, LANES)]
            return 0

        lax.fori_loop(0, DW // LANES, addp, 0)


def _zero_acc(accS, dbuf, s):
    def zd(k, _):
        dbuf[k >> 3, pl.ds((k & 7) * LANES, LANES)] = jnp.zeros((LANES,), _F32)
        return 0

    lax.fori_loop(0, 80 * 8, zd, 0)
    for j in range(CHK // 80):
        pltpu.sync_copy(dbuf, accS.at[pl.ds(s * CHK + j * 80, 80)])


def _alpha_batch(lgv, dstv, denv, alv, g, B):
    for gi in range(B // LANES):
        sl = pl.ds(gi * LANES, LANES)
        ex = jnp.exp(lgv[sl] - g)
        d = dstv[sl]
        den = plsc.load_gather(denv, [d])
        alv[sl] = ex / (den + 1e-16)


def _scale_rows(rows, alv, B):
    def sc(gi, _):
        av = alv[pl.ds(gi * LANES, LANES)]
        for j in range(LANES):
            i = gi * LANES + j
            a = av[j]
            for kk in range(8):
                sl = pl.ds(kk * LANES, LANES)
                rows[i, sl] = rows[i, sl] * a
        return 0

    lax.fori_loop(0, B // LANES, sc, 0)


def _s3_split_body(srch, dsth, lgh, wmh, dph, xla, xlb, bch, outa, outb,
                   srcv, dstv, lgv, alv, rows, denv, dtmp, wmv, bcv, dbuf,
                   accS, sem):
    B = 80
    EC = E // NS
    c = lax.axis_index("c")
    s = lax.axis_index("s")
    base = s * EC

    pltpu.sync_copy(wmh, wmv)
    g = _gmax(wmv)
    _denom_combine(dph, denv, dtmp)
    pltpu.sync_copy(bch.at[c], bcv)
    _zero_acc(accS, dbuf, s)
    plsc.subcore_barrier()

    def batch(b, _):
        off = base + b * B
        pltpu.sync_copy(srch.at[pl.ds(off, B)], srcv)
        pltpu.sync_copy(dsth.at[pl.ds(off, B)], dstv)
        pltpu.sync_copy(lgh.at[pl.ds(off, B)], lgv)

        @pl.when(c == 0)
        def _():
            pltpu.async_copy(xla.at[srcv], rows, sem).wait()

        @pl.when(c == 1)
        def _():
            pltpu.async_copy(xlb.at[srcv], rows, sem).wait()

        _alpha_batch(lgv, dstv, denv, alv, g, B)
        _scale_rows(rows, alv, B)
        pltpu.sync_copy(rows, accS.at[dstv], add=True)
        return 0

    lax.fori_loop(0, EC // B, batch, 0)
    plsc.subcore_barrier()

    for j in range(CHK // 80):
        r0 = s * CHK + j * 80
        pltpu.sync_copy(accS.at[pl.ds(r0, 80)], dbuf)

        def br(i, _):
            for kk in range(8):
                sl = pl.ds(kk * LANES, LANES)
                dbuf[i, sl] = jnp.maximum(dbuf[i, sl] + bcv[sl], 0.0)
            return 0

        lax.fori_loop(0, 80, br, 0)

        @pl.when(c == 0)
        def _():
            pltpu.sync_copy(dbuf, outa.at[pl.ds(r0, 80)])

        @pl.when(c == 1)
        def _():
            pltpu.sync_copy(dbuf, outb.at[pl.ds(r0, 80)])


def _s3_split(src, dst, lg, wm, dp, xla, xlb, bc1r):
    B = 80
    return pl.kernel(
        _s3_split_body,
        out_type=[jax.ShapeDtypeStruct((NPS, 128), _F32)] * 2,
        mesh=_MESH,
        compiler_params=pltpu.CompilerParams(needs_layout_passes=False),
        scratch_types=[pltpu.VMEM((B,), _I32), pltpu.VMEM((B,), _I32),
                       pltpu.VMEM((B,), _F32), pltpu.VMEM((B,), _F32),
                       pltpu.VMEM((B, 128), _F32),
                       pltpu.VMEM((DH * DW,), _F32), pltpu.VMEM((DW,), _F32),
                       pltpu.VMEM((NW, LANES), _F32), pltpu.VMEM((128,), _F32),
                       pltpu.VMEM((80, 128), _F32),
                       pltpu.VMEM_SHARED((NPS, 128), _F32),
                       pltpu.SemaphoreType.DMA],
    )(src, dst, lg, wm, dp, xla, xlb, bc1r)


def _s3_full_body(srch, dsth, lgh, wmh, dph, xlh, p0h, p1h,
                  srcv, dstv, lgv, alv, rows, denv, dtmp, wmv, dbuf,
                  accS, sem):
    B = 80
    EC = E // NW
    c = lax.axis_index("c")
    s = lax.axis_index("s")
    base = c * (E // NC) + s * EC

    pltpu.sync_copy(wmh, wmv)
    g = _gmax(wmv)
    _denom_combine(dph, denv, dtmp)
    _zero_acc(accS, dbuf, s)
    plsc.subcore_barrier()

    def batch(b, _):
        off = base + b * B
        pltpu.sync_copy(srch.at[pl.ds(off, B)], srcv)
        pltpu.sync_copy(dsth.at[pl.ds(off, B)], dstv)
        pltpu.sync_copy(lgh.at[pl.ds(off, B)], lgv)
        pltpu.async_copy(xlh.at[srcv], rows, sem).wait()
        _alpha_batch(lgv, dstv, denv, alv, g, B)
        _scale_rows(rows, alv, B)
        pltpu.sync_copy(rows, accS.at[dstv], add=True)
        return 0

    lax.fori_loop(0, EC // B, batch, 0)
    plsc.subcore_barrier()

    for j in range(CHK // 80):
        r0 = s * CHK + j * 80
        pltpu.sync_copy(accS.at[pl.ds(r0, 80)], dbuf)

        @pl.when(c == 0)
        def _():
            pltpu.sync_copy(dbuf, p0h.at[pl.ds(r0, 80)])

        @pl.when(c == 1)
        def _():
            pltpu.sync_copy(dbuf, p1h.at[pl.ds(r0, 80)])


def _s3_full(src, dst, lg, wm, dp, xl2):
    B = 80
    return pl.kernel(
        _s3_full_body,
        out_type=[jax.ShapeDtypeStruct((NPS, 128), _F32)] * 2,
        mesh=_MESH,
        compiler_params=pltpu.CompilerParams(needs_layout_passes=False),
        scratch_types=[pltpu.VMEM((B,), _I32), pltpu.VMEM((B,), _I32),
                       pltpu.VMEM((B,), _F32), pltpu.VMEM((B,), _F32),
                       pltpu.VMEM((B, 128), _F32),
                       pltpu.VMEM((DH * DW,), _F32), pltpu.VMEM((DW,), _F32),
                       pltpu.VMEM((NW, LANES), _F32),
                       pltpu.VMEM((80, 128), _F32),
                       pltpu.VMEM_SHARED((NPS, 128), _F32),
                       pltpu.SemaphoreType.DMA],
    )(src, dst, lg, wm, dp, xl2)


def _s4_body(zuh, zvh, e0h, e1h, b1h, h1h, uv, vv, bufu, bufv, b1v, sem):
    B = 128
    EW = EP2 // NW
    wid = lax.axis_index("s") * NC + lax.axis_index("c")
    base = wid * EW
    pltpu.sync_copy(b1h, b1v)

    def batch(b, _):
        off = base + b * B
        pltpu.sync_copy(e0h.at[pl.ds(off, B)], uv)
        pltpu.sync_copy(e1h.at[pl.ds(off, B)], vv)
        d1 = pltpu.async_copy(zuh.at[uv], bufu, sem)
        d2 = pltpu.async_copy(zvh.at[vv], bufv, sem)
        d1.wait(); d2.wait()

        def row(i, _2):
            for kk in range(8):
                sl = pl.ds(kk * LANES, LANES)
                bufu[i, sl] = jnp.maximum(bufu[i, sl] + bufv[i, sl] + b1v[sl], 0.0)
            return 0

        lax.fori_loop(0, B, row, 0)
        pltpu.sync_copy(bufu, h1h.at[pl.ds(off, B)])
        return 0

    lax.fori_loop(0, EW // B, batch, 0)


def _s4(ZU, ZV, ei0, ei1, b1):
    B = 128
    return pl.kernel(
        _s4_body,
        out_type=jax.ShapeDtypeStruct((EP2, 128), _F32),
        mesh=_MESH,
        compiler_params=pltpu.CompilerParams(needs_layout_passes=False),
        scratch_types=[pltpu.VMEM((B,), _I32), pltpu.VMEM((B,), _I32),
                       pltpu.VMEM((B, 128), _F32), pltpu.VMEM((B, 128), _F32),
                       pltpu.VMEM((128,), _F32), pltpu.SemaphoreType.DMA],
    )(ZU, ZV, ei0, ei1, b1)


def kernel(x, edge_index, pos_edge_index, neg_edge_index,
           Wl1, Wr1, att1, bc1, Wl2, Wr2, att2, bc2,
           W1, b1, W2, b2, W3, b3, W4, b4):
    src = edge_index[0]
    dst = edge_index[1]

    xla, xlb, xra, xrb = _m1(x, Wl1, Wr1)
    lg1, wm1 = _s1_split(xla, xlb, xra, xrb, att1, src, dst)
    dp1 = _s2(lg1, dst, wm1)
    z1a, z1b = _s3_split(src, dst, lg1, wm1, dp1, xla, xlb,
                         bc1.reshape(2, 128))

    xl2, xr2 = _m2(z1a[:N], z1b[:N], Wl2, Wr2)
    lg2, wm2 = _s1_full(xl2, xr2, att2, src, dst)
    dp2 = _s2(lg2, dst, wm2)
    p0, p1 = _s3_full(src, dst, lg2, wm2, dp2, xl2)

    ZU, ZV = _m3(p0[:N], p1[:N], bc2, W1)
    ei0 = jnp.concatenate([pos_edge_index[0], neg_edge_index[0]])
    ei1 = jnp.concatenate([pos_edge_index[1], neg_edge_index[1]])
    h1 = _s4(ZU, ZV, ei0, ei1, b1)
    return _m4(h1, W2, b2, W3, b3, W4, b4)


# speedup vs baseline: 5.4547x; 1.0555x over previous
import jax
import jax.numpy as jnp
from jax import lax
from jax.experimental import pallas as pl
from jax.experimental.pallas import tpu as pltpu
from jax.experimental.pallas import tpu_sc as plsc

N = 10000
E = 320000
EP2 = 65536
NC, NS, LANES = 2, 16, 16
NW = NC * NS
NPS = 10240
CHK = NPS // NS
DH, DW = 16, 1024

_MESH = plsc.VectorSubcoreMesh(core_axis_name="c", subcore_axis_name="s")
_F32 = jnp.float32
_I32 = jnp.int32


def _m1_body(x_ref, wl_ref, wr_ref, xla, xlb, xra, xrb):
    xw = jnp.dot(x_ref[...], wl_ref[...], preferred_element_type=_F32)
    xv = jnp.dot(x_ref[...], wr_ref[...], preferred_element_type=_F32)
    xla[...] = xw[:, :128]
    xlb[...] = xw[:, 128:]
    xra[...] = xv[:, :128]
    xrb[...] = xv[:, 128:]


def _m1(x, Wl1, Wr1):
    BM = 1000
    return pl.pallas_call(
        _m1_body,
        grid=(N // BM,),
        in_specs=[pl.BlockSpec((BM, 128), lambda m: (m, 0)),
                  pl.BlockSpec((128, 256), lambda m: (0, 0)),
                  pl.BlockSpec((128, 256), lambda m: (0, 0))],
        out_specs=[pl.BlockSpec((BM, 128), lambda m: (m, 0))] * 4,
        out_shape=[jax.ShapeDtypeStruct((N, 128), _F32)] * 4,
    )(x, Wl1, Wr1)


def _m2_body(za_ref, zb_ref, wl_ref, wr_ref, xl2, xr2):
    za = za_ref[...]
    zb = zb_ref[...]
    xl2[...] = (jnp.dot(za, wl_ref[...][:128, :], preferred_element_type=_F32)
                + jnp.dot(zb, wl_ref[...][128:, :], preferred_element_type=_F32))
    xr2[...] = (jnp.dot(za, wr_ref[...][:128, :], preferred_element_type=_F32)
                + jnp.dot(zb, wr_ref[...][128:, :], preferred_element_type=_F32))


def _m2(z1a, z1b, Wl2, Wr2):
    BM = 1000
    return pl.pallas_call(
        _m2_body,
        grid=(N // BM,),
        in_specs=[pl.BlockSpec((BM, 128), lambda m: (m, 0)),
                  pl.BlockSpec((BM, 128), lambda m: (m, 0)),
                  pl.BlockSpec((256, 128), lambda m: (0, 0)),
                  pl.BlockSpec((256, 128), lambda m: (0, 0))],
        out_specs=[pl.BlockSpec((BM, 128), lambda m: (m, 0))] * 2,
        out_shape=[jax.ShapeDtypeStruct((N, 128), _F32)] * 2,
    )(z1a, z1b, Wl2, Wr2)


def _m3_body(p0_ref, p1_ref, bc_ref, w1_ref, zu, zv):
    z2 = p0_ref[...] + p1_ref[...] + bc_ref[...][None, :]
    zu[...] = jnp.dot(z2, w1_ref[...][:128, :], preferred_element_type=_F32)
    zv[...] = jnp.dot(z2, w1_ref[...][128:, :], preferred_element_type=_F32)


def _m3(p0, p1, bc2, W1):
    BM = 1000
    return pl.pallas_call(
        _m3_body,
        grid=(N // BM,),
        in_specs=[pl.BlockSpec((BM, 128), lambda m: (m, 0)),
                  pl.BlockSpec((BM, 128), lambda m: (m, 0)),
                  pl.BlockSpec((128,), lambda m: (0,)),
                  pl.BlockSpec((256, 128), lambda m: (0, 0))],
        out_specs=[pl.BlockSpec((BM, 128), lambda m: (m, 0))] * 2,
        out_shape=[jax.ShapeDtypeStruct((N, 128), _F32)] * 2,
    )(p0, p1, bc2, W1)


def _m4_body(h_ref, w2_ref, b2_ref, w3_ref, b3_ref, w4_ref, b4_ref, out):
    h2 = jnp.maximum(jnp.dot(h_ref[...], w2_ref[...],
                             preferred_element_type=_F32) + b2_ref[...][None, :], 0.0)
    h3 = jnp.maximum(jnp.dot(h2, w3_ref[...],
                             preferred_element_type=_F32) + b3_ref[...][None, :], 0.0)
    out[...] = jnp.dot(h3, w4_ref[...], preferred_element_type=_F32) + b4_ref[...]


def _m4(h1, W2, b2, W3, b3, W4, b4):
    BM = 2048
    o = pl.pallas_call(
        _m4_body,
        grid=(EP2 // BM,),
        in_specs=[pl.BlockSpec((BM, 128), lambda m: (m, 0)),
                  pl.BlockSpec((128, 128), lambda m: (0, 0)),
                  pl.BlockSpec((128,), lambda m: (0,)),
                  pl.BlockSpec((128, 64), lambda m: (0, 0)),
                  pl.BlockSpec((64,), lambda m: (0,)),
                  pl.BlockSpec((64, 1), lambda m: (0, 0)),
                  pl.BlockSpec((1,), lambda m: (0,))],
        out_specs=pl.BlockSpec((BM, 1), lambda m: (m, 0)),
        out_shape=jax.ShapeDtypeStruct((EP2, 1), _F32),
    )(h1, W2, b2, W3, b3, W4, b4)
    return jnp.squeeze(o, axis=1)


def _edge_logits_groups(pairs, attv, outv, maxv, tbuf, B):
    lane17 = lax.iota(_I32, LANES) * 17

    def group(g, _):
        for j in range(LANES):
            i = g * LANES + j
            acc = jnp.zeros((LANES,), _F32)
            for (bl, br, off) in pairs:
                for fc in range(128 // LANES):
                    sl = pl.ds(fc * LANES, LANES)
                    u = bl[i, sl] + br[i, sl]
                    t = jnp.maximum(u, 0.2 * u)
                    acc = acc + t * attv[pl.ds(off + fc * LANES, LANES)]
            tbuf[pl.ds(j * 17, LANES)] = acc
        gacc = plsc.load_gather(tbuf, [lane17])
        for k in range(1, LANES):
            gacc = gacc + plsc.load_gather(tbuf, [lane17 + k])
        outv[pl.ds(g * LANES, LANES)] = gacc
        maxv[...] = jnp.maximum(maxv[...], gacc)
        return 0

    lax.fori_loop(0, B // LANES, group, 0)


def _s1_split_body(xla, xlb, xra, xrb, atth, srch, dsth, lgh, wmh,
                   srcv, dstv, bla, blb, bra, brb, attv, outv, maxv, tbuf,
                   sem):
    B = 80
    EW = E // NW
    wid = lax.axis_index("s") * NC + lax.axis_index("c")
    base = wid * EW
    pltpu.sync_copy(atth, attv)
    maxv[...] = jnp.full((LANES,), -1e30, _F32)

    def batch(b, carry):
        off = base + b * B
        pltpu.sync_copy(srch.at[pl.ds(off, B)], srcv)
        pltpu.sync_copy(dsth.at[pl.ds(off, B)], dstv)
        d1 = pltpu.async_copy(xla.at[srcv], bla, sem)
        d2 = pltpu.async_copy(xlb.at[srcv], blb, sem)
        d3 = pltpu.async_copy(xra.at[dstv], bra, sem)
        d4 = pltpu.async_copy(xrb.at[dstv], brb, sem)
        d1.wait(); d2.wait(); d3.wait(); d4.wait()
        _edge_logits_groups([(bla, bra, 0), (blb, brb, 128)], attv, outv,
                            maxv, tbuf, B)
        pltpu.sync_copy(outv, lgh.at[pl.ds(off, B)])
        return carry

    lax.fori_loop(0, EW // B, batch, 0)
    pltpu.sync_copy(maxv, wmh.at[wid])


def _s1_split(xla, xlb, xra, xrb, att1, src, dst):
    B = 80
    return pl.kernel(
        _s1_split_body,
        out_type=[jax.ShapeDtypeStruct((E,), _F32),
                  jax.ShapeDtypeStruct((NW, LANES), _F32)],
        mesh=_MESH,
        compiler_params=pltpu.CompilerParams(needs_layout_passes=False),
        scratch_types=[pltpu.VMEM((B,), _I32), pltpu.VMEM((B,), _I32),
                       pltpu.VMEM((B, 128), _F32), pltpu.VMEM((B, 128), _F32),
                       pltpu.VMEM((B, 128), _F32), pltpu.VMEM((B, 128), _F32),
                       pltpu.VMEM((256,), _F32), pltpu.VMEM((B,), _F32),
                       pltpu.VMEM((LANES,), _F32),
                       pltpu.VMEM((17 * LANES,), _F32),
                       pltpu.SemaphoreType.DMA],
    )(xla, xlb, xra, xrb, att1, src, dst)


def _s1_full_body(xlh, xrh, atth, srch, dsth, lgh, wmh,
                  srcv, dstv, bl, br, attv, outv, maxv, tbuf, sem):
    B = 80
    K = 5
    EW = E // NW
    wid = lax.axis_index("s") * NC + lax.axis_index("c")
    base = wid * EW
    pltpu.sync_copy(atth, attv)
    maxv[...] = jnp.full((LANES,), -1e30, _F32)

    def sup(go, carry):
        off = base + go * (K * B)
        ds = []
        for b in range(K):
            pltpu.sync_copy(srch.at[pl.ds(off + b * B, B)], srcv.at[b])
            pltpu.sync_copy(dsth.at[pl.ds(off + b * B, B)], dstv.at[b])
            d1 = pltpu.async_copy(xlh.at[srcv.at[b]], bl.at[b], sem.at[b])
            d2 = pltpu.async_copy(xrh.at[dstv.at[b]], br.at[b], sem.at[b])
            ds.append((d1, d2))
        for b in range(K):
            d1, d2 = ds[b]
            d1.wait(); d2.wait()
            _edge_logits_groups([(bl.at[b], br.at[b], 0)], attv, outv,
                                maxv, tbuf, B)
            pltpu.sync_copy(outv, lgh.at[pl.ds(off + b * B, B)])
        return carry

    lax.fori_loop(0, EW // (K * B), sup, 0)
    pltpu.sync_copy(maxv, wmh.at[wid])


def _s1_full(xl2, xr2, att2, src, dst):
    B = 80
    K = 5
    return pl.kernel(
        _s1_full_body,
        out_type=[jax.ShapeDtypeStruct((E,), _F32),
                  jax.ShapeDtypeStruct((NW, LANES), _F32)],
        mesh=_MESH,
        compiler_params=pltpu.CompilerParams(needs_layout_passes=False),
        scratch_types=[pltpu.VMEM((K, B), _I32), pltpu.VMEM((K, B), _I32),
                       pltpu.VMEM((K, B, 128), _F32),
                       pltpu.VMEM((K, B, 128), _F32),
                       pltpu.VMEM((128,), _F32), pltpu.VMEM((B,), _F32),
                       pltpu.VMEM((LANES,), _F32),
                       pltpu.VMEM((17 * LANES,), _F32),
                       pltpu.SemaphoreType.DMA((K,))],
    )(xl2, xr2, att2, src, dst)


def _gmax(wmv):
    m = wmv[0]

    def mx(k, mm):
        return jnp.maximum(mm, wmv[k])

    m = lax.fori_loop(1, NW, mx, m)
    g = m[0]
    for k in range(1, LANES):
        g = jnp.maximum(g, m[k])
    return g


def _s2_body(lgh, dsth, wmh, dph, lgv, dstv, wmv, denl, sbuf, resv, shared):
    B2 = 2000
    EW = E // NW
    c = lax.axis_index("c")
    s = lax.axis_index("s")
    wid = s * NC + c
    base = wid * EW

    def zz(k, _):
        denl[pl.ds(k * LANES, LANES)] = jnp.zeros((LANES,), _F32)
        return 0

    lax.fori_loop(0, (DH * DW) // LANES, zz, 0)

    pltpu.sync_copy(wmh, wmv)
    g = _gmax(wmv)

    def batch(b, _):
        off = base + b * B2
        pltpu.sync_copy(lgh.at[pl.ds(off, B2)], lgv)
        pltpu.sync_copy(dsth.at[pl.ds(off, B2)], dstv)

        def grp(k, _2):
            sl = pl.ds(k * LANES, LANES)
            ex = jnp.exp(lgv[sl] - g)
            d = dstv[sl]
            plsc.addupdate_scatter(denl, [d], ex)
            return 0

        lax.fori_loop(0, B2 // LANES, grp, 0)
        return 0

    lax.fori_loop(0, EW // B2, batch, 0)

    for t in range(NS):
        pltpu.sync_copy(denl.at[pl.ds(t * DW, DW)], shared.at[t, s])
    plsc.subcore_barrier()
    pltpu.sync_copy(shared.at[s], sbuf)

    def col(k, _):
        sl = pl.ds(k * LANES, LANES)
        a = sbuf[0, sl]
        for p in range(1, NS):
            a = a + sbuf[p, sl]
        resv[sl] = a
        return 0

    lax.fori_loop(0, DW // LANES, col, 0)
    pltpu.sync_copy(resv, dph.at[c, pl.ds(s * DW, DW)])


def _s2(lg, dst, wm):
    return pl.kernel(
        _s2_body,
        out_type=jax.ShapeDtypeStruct((NC, DH * DW), _F32),
        mesh=_MESH,
        compiler_params=pltpu.CompilerParams(needs_layout_passes=False),
        scratch_types=[pltpu.VMEM((2000,), _F32), pltpu.VMEM((2000,), _I32),
                       pltpu.VMEM((NW, LANES), _F32),
                       pltpu.VMEM((DH * DW,), _F32),
                       pltpu.VMEM((NS, DW), _F32), pltpu.VMEM((DW,), _F32),
                       pltpu.VMEM_SHARED((NS, NS, DW), _F32)],
    )(lg, dst, wm)


def _denom_combine(dph, denv, dtmp):
    pltpu.sync_copy(dph.at[0], denv)
    for t in range(DH):
        pltpu.sync_copy(dph.at[1, pl.ds(t * DW, DW)], dtmp)

        def addp(k, _):
            sl = pl.ds(t * DW + k * LANES, LANES)
            denv[sl] = denv[sl] + dtmp[pl.ds(k * LANES, LANES)]
            return 0

        lax.fori_loop(0, DW // LANES, addp, 0)


def _zero_acc(accS, dbuf, s):
    def zd(k, _):
        dbuf[k >> 3, pl.ds((k & 7) * LANES, LANES)] = jnp.zeros((LANES,), _F32)
        return 0

    lax.fori_loop(0, 80 * 8, zd, 0)
    for j in range(CHK // 80):
        pltpu.sync_copy(dbuf, accS.at[pl.ds(s * CHK + j * 80, 80)])


def _alpha_batch(lgv, dstv, denv, alv, g, B):
    for gi in range(B // LANES):
        sl = pl.ds(gi * LANES, LANES)
        ex = jnp.exp(lgv[sl] - g)
        d = dstv[sl]
        den = plsc.load_gather(denv, [d])
        alv[sl] = ex / (den + 1e-16)


def _scale_rows(rows, alv, B):
    def sc(gi, _):
        av = alv[pl.ds(gi * LANES, LANES)]
        for j in range(LANES):
            i = gi * LANES + j
            a = av[j]
            for kk in range(8):
                sl = pl.ds(kk * LANES, LANES)
                rows[i, sl] = rows[i, sl] * a
        return 0

    lax.fori_loop(0, B // LANES, sc, 0)


def _s3_split_body(srch, dsth, lgh, wmh, dph, xla, xlb, bch, outa, outb,
                   srcv, dstv, lgv, alv, rows, denv, dtmp, wmv, bcv, dbuf,
                   accS, sem):
    B = 80
    EC = E // NS
    c = lax.axis_index("c")
    s = lax.axis_index("s")
    base = s * EC

    pltpu.sync_copy(wmh, wmv)
    g = _gmax(wmv)
    _denom_combine(dph, denv, dtmp)
    pltpu.sync_copy(bch.at[c], bcv)
    _zero_acc(accS, dbuf, s)
    plsc.subcore_barrier()

    def batch(b, _):
        off = base + b * B
        pltpu.sync_copy(srch.at[pl.ds(off, B)], srcv)
        pltpu.sync_copy(dsth.at[pl.ds(off, B)], dstv)
        pltpu.sync_copy(lgh.at[pl.ds(off, B)], lgv)

        @pl.when(c == 0)
        def _():
            pltpu.async_copy(xla.at[srcv], rows, sem).wait()

        @pl.when(c == 1)
        def _():
            pltpu.async_copy(xlb.at[srcv], rows, sem).wait()

        _alpha_batch(lgv, dstv, denv, alv, g, B)
        _scale_rows(rows, alv, B)
        pltpu.sync_copy(rows, accS.at[dstv], add=True)
        return 0

    lax.fori_loop(0, EC // B, batch, 0)
    plsc.subcore_barrier()

    for j in range(CHK // 80):
        r0 = s * CHK + j * 80
        pltpu.sync_copy(accS.at[pl.ds(r0, 80)], dbuf)

        def br(i, _):
            for kk in range(8):
                sl = pl.ds(kk * LANES, LANES)
                dbuf[i, sl] = jnp.maximum(dbuf[i, sl] + bcv[sl], 0.0)
            return 0

        lax.fori_loop(0, 80, br, 0)

        @pl.when(c == 0)
        def _():
            pltpu.sync_copy(dbuf, outa.at[pl.ds(r0, 80)])

        @pl.when(c == 1)
        def _():
            pltpu.sync_copy(dbuf, outb.at[pl.ds(r0, 80)])


def _s3_split(src, dst, lg, wm, dp, xla, xlb, bc1r):
    B = 80
    return pl.kernel(
        _s3_split_body,
        out_type=[jax.ShapeDtypeStruct((NPS, 128), _F32)] * 2,
        mesh=_MESH,
        compiler_params=pltpu.CompilerParams(needs_layout_passes=False),
        scratch_types=[pltpu.VMEM((B,), _I32), pltpu.VMEM((B,), _I32),
                       pltpu.VMEM((B,), _F32), pltpu.VMEM((B,), _F32),
                       pltpu.VMEM((B, 128), _F32),
                       pltpu.VMEM((DH * DW,), _F32), pltpu.VMEM((DW,), _F32),
                       pltpu.VMEM((NW, LANES), _F32), pltpu.VMEM((128,), _F32),
                       pltpu.VMEM((80, 128), _F32),
                       pltpu.VMEM_SHARED((NPS, 128), _F32),
                       pltpu.SemaphoreType.DMA],
    )(src, dst, lg, wm, dp, xla, xlb, bc1r)


def _s3_full_body(srch, dsth, lgh, wmh, dph, xlh, p0h, p1h,
                  srcv, dstv, lgv, alv, rows, denv, dtmp, wmv, dbuf,
                  accS, sem):
    B = 80
    EC = E // NW
    c = lax.axis_index("c")
    s = lax.axis_index("s")
    base = c * (E // NC) + s * EC

    pltpu.sync_copy(wmh, wmv)
    g = _gmax(wmv)
    _denom_combine(dph, denv, dtmp)
    _zero_acc(accS, dbuf, s)
    plsc.subcore_barrier()

    def batch(b, _):
        off = base + b * B
        pltpu.sync_copy(srch.at[pl.ds(off, B)], srcv)
        pltpu.sync_copy(dsth.at[pl.ds(off, B)], dstv)
        pltpu.sync_copy(lgh.at[pl.ds(off, B)], lgv)
        pltpu.async_copy(xlh.at[srcv], rows, sem).wait()
        _alpha_batch(lgv, dstv, denv, alv, g, B)
        _scale_rows(rows, alv, B)
        pltpu.sync_copy(rows, accS.at[dstv], add=True)
        return 0

    lax.fori_loop(0, EC // B, batch, 0)
    plsc.subcore_barrier()

    for j in range(CHK // 80):
        r0 = s * CHK + j * 80
        pltpu.sync_copy(accS.at[pl.ds(r0, 80)], dbuf)

        @pl.when(c == 0)
        def _():
            pltpu.sync_copy(dbuf, p0h.at[pl.ds(r0, 80)])

        @pl.when(c == 1)
        def _():
            pltpu.sync_copy(dbuf, p1h.at[pl.ds(r0, 80)])


def _s3_full(src, dst, lg, wm, dp, xl2):
    B = 80
    return pl.kernel(
        _s3_full_body,
        out_type=[jax.ShapeDtypeStruct((NPS, 128), _F32)] * 2,
        mesh=_MESH,
        compiler_params=pltpu.CompilerParams(needs_layout_passes=False),
        scratch_types=[pltpu.VMEM((B,), _I32), pltpu.VMEM((B,), _I32),
                       pltpu.VMEM((B,), _F32), pltpu.VMEM((B,), _F32),
                       pltpu.VMEM((B, 128), _F32),
                       pltpu.VMEM((DH * DW,), _F32), pltpu.VMEM((DW,), _F32),
                       pltpu.VMEM((NW, LANES), _F32),
                       pltpu.VMEM((80, 128), _F32),
                       pltpu.VMEM_SHARED((NPS, 128), _F32),
                       pltpu.SemaphoreType.DMA],
    )(src, dst, lg, wm, dp, xl2)


def _s4_body(zuh, zvh, e0h, e1h, b1h, h1h, uv, vv, bufu, bufv, b1v, sem):
    B = 128
    EW = EP2 // NW
    wid = lax.axis_index("s") * NC + lax.axis_index("c")
    base = wid * EW
    pltpu.sync_copy(b1h, b1v)

    def batch(b, _):
        off = base + b * B
        pltpu.sync_copy(e0h.at[pl.ds(off, B)], uv)
        pltpu.sync_copy(e1h.at[pl.ds(off, B)], vv)
        d1 = pltpu.async_copy(zuh.at[uv], bufu, sem)
        d2 = pltpu.async_copy(zvh.at[vv], bufv, sem)
        d1.wait(); d2.wait()

        def row(i, _2):
            for kk in range(8):
                sl = pl.ds(kk * LANES, LANES)
                bufu[i, sl] = jnp.maximum(bufu[i, sl] + bufv[i, sl] + b1v[sl], 0.0)
            return 0

        lax.fori_loop(0, B, row, 0)
        pltpu.sync_copy(bufu, h1h.at[pl.ds(off, B)])
        return 0

    lax.fori_loop(0, EW // B, batch, 0)


def _s4(ZU, ZV, ei0, ei1, b1):
    B = 128
    return pl.kernel(
        _s4_body,
        out_type=jax.ShapeDtypeStruct((EP2, 128), _F32),
        mesh=_MESH,
        compiler_params=pltpu.CompilerParams(needs_layout_passes=False),
        scratch_types=[pltpu.VMEM((B,), _I32), pltpu.VMEM((B,), _I32),
                       pltpu.VMEM((B, 128), _F32), pltpu.VMEM((B, 128), _F32),
                       pltpu.VMEM((128,), _F32), pltpu.SemaphoreType.DMA],
    )(ZU, ZV, ei0, ei1, b1)


def kernel(x, edge_index, pos_edge_index, neg_edge_index,
           Wl1, Wr1, att1, bc1, Wl2, Wr2, att2, bc2,
           W1, b1, W2, b2, W3, b3, W4, b4):
    src = edge_index[0]
    dst = edge_index[1]

    xla, xlb, xra, xrb = _m1(x, Wl1, Wr1)
    lg1, wm1 = _s1_split(xla, xlb, xra, xrb, att1, src, dst)
    dp1 = _s2(lg1, dst, wm1)
    z1a, z1b = _s3_split(src, dst, lg1, wm1, dp1, xla, xlb,
                         bc1.reshape(2, 128))

    xl2, xr2 = _m2(z1a[:N], z1b[:N], Wl2, Wr2)
    lg2, wm2 = _s1_full(xl2, xr2, att2, src, dst)
    dp2 = _s2(lg2, dst, wm2)
    p0, p1 = _s3_full(src, dst, lg2, wm2, dp2, xl2)

    ZU, ZV = _m3(p0[:N], p1[:N], bc2, W1)
    ei0 = jnp.concatenate([pos_edge_index[0], neg_edge_index[0]])
    ei1 = jnp.concatenate([pos_edge_index[1], neg_edge_index[1]])
    h1 = _s4(ZU, ZV, ei0, ei1, b1)
    return _m4(h1, W2, b2, W3, b3, W4, b4)


# speedup vs baseline: 6.5093x; 1.1934x over previous
import jax
import jax.numpy as jnp
from jax import lax
from jax.experimental import pallas as pl
from jax.experimental.pallas import tpu as pltpu
from jax.experimental.pallas import tpu_sc as plsc

N = 10000
E = 320000
EP2 = 65536
NC, NS, LANES = 2, 16, 16
NW = NC * NS
NPS = 10240
CHK = NPS // NS
DH, DW = 16, 1024

_MESH = plsc.VectorSubcoreMesh(core_axis_name="c", subcore_axis_name="s")
_F32 = jnp.float32
_I32 = jnp.int32


def _m1_body(x_ref, wl_ref, wr_ref, xl, xr, xstk):
    xw = jnp.dot(x_ref[...], wl_ref[...], preferred_element_type=_F32)
    xv = jnp.dot(x_ref[...], wr_ref[...], preferred_element_type=_F32)
    xl[...] = xw
    xr[...] = xv
    xstk[...] = xw


def _m1(x, Wl1, Wr1):
    BM = 1000
    return pl.pallas_call(
        _m1_body,
        grid=(2, N // BM),
        in_specs=[pl.BlockSpec((BM, 128), lambda h, m: (m, 0)),
                  pl.BlockSpec((128, 128), lambda h, m: (0, h)),
                  pl.BlockSpec((128, 128), lambda h, m: (0, h))],
        out_specs=[pl.BlockSpec((BM, 128), lambda h, m: (m, h)),
                   pl.BlockSpec((BM, 128), lambda h, m: (m, h)),
                   pl.BlockSpec((BM, 128),
                                lambda h, m: (h * (N // BM) + m, 0))],
        out_shape=[jax.ShapeDtypeStruct((N, 256), _F32),
                   jax.ShapeDtypeStruct((N, 256), _F32),
                   jax.ShapeDtypeStruct((2 * N, 128), _F32)],
    )(x, Wl1, Wr1)


def _m2_body(za_ref, zb_ref, wl_ref, wr_ref, xl2, xr2):
    za = za_ref[...]
    zb = zb_ref[...]
    xl2[...] = (jnp.dot(za, wl_ref[...][:128, :], preferred_element_type=_F32)
                + jnp.dot(zb, wl_ref[...][128:, :], preferred_element_type=_F32))
    xr2[...] = (jnp.dot(za, wr_ref[...][:128, :], preferred_element_type=_F32)
                + jnp.dot(zb, wr_ref[...][128:, :], preferred_element_type=_F32))


def _m2(z1a, z1b, Wl2, Wr2):
    BM = 1000
    return pl.pallas_call(
        _m2_body,
        grid=(N // BM,),
        in_specs=[pl.BlockSpec((BM, 128), lambda m: (m, 0)),
                  pl.BlockSpec((BM, 128), lambda m: (m, 0)),
                  pl.BlockSpec((256, 128), lambda m: (0, 0)),
                  pl.BlockSpec((256, 128), lambda m: (0, 0))],
        out_specs=[pl.BlockSpec((BM, 128), lambda m: (m, 0))] * 2,
        out_shape=[jax.ShapeDtypeStruct((N, 128), _F32)] * 2,
    )(z1a, z1b, Wl2, Wr2)


def _m3_body(p0_ref, p1_ref, bc_ref, w1_ref, zu, zv):
    z2 = p0_ref[...] + p1_ref[...] + bc_ref[...][None, :]
    zu[...] = jnp.dot(z2, w1_ref[...][:128, :], preferred_element_type=_F32)
    zv[...] = jnp.dot(z2, w1_ref[...][128:, :], preferred_element_type=_F32)


def _m3(p0, p1, bc2, W1):
    BM = 1000
    return pl.pallas_call(
        _m3_body,
        grid=(N // BM,),
        in_specs=[pl.BlockSpec((BM, 128), lambda m: (m, 0)),
                  pl.BlockSpec((BM, 128), lambda m: (m, 0)),
                  pl.BlockSpec((128,), lambda m: (0,)),
                  pl.BlockSpec((256, 128), lambda m: (0, 0))],
        out_specs=[pl.BlockSpec((BM, 128), lambda m: (m, 0))] * 2,
        out_shape=[jax.ShapeDtypeStruct((N, 128), _F32)] * 2,
    )(p0, p1, bc2, W1)


def _m4_body(h_ref, w2_ref, b2_ref, w3_ref, b3_ref, w4_ref, b4_ref, out):
    h2 = jnp.maximum(jnp.dot(h_ref[...], w2_ref[...],
                             preferred_element_type=_F32) + b2_ref[...][None, :], 0.0)
    h3 = jnp.maximum(jnp.dot(h2, w3_ref[...],
                             preferred_element_type=_F32) + b3_ref[...][None, :], 0.0)
    out[...] = jnp.dot(h3, w4_ref[...], preferred_element_type=_F32) + b4_ref[...]


def _m4(h1, W2, b2, W3, b3, W4, b4):
    BM = 2048
    o = pl.pallas_call(
        _m4_body,
        grid=(EP2 // BM,),
        in_specs=[pl.BlockSpec((BM, 128), lambda m: (m, 0)),
                  pl.BlockSpec((128, 128), lambda m: (0, 0)),
                  pl.BlockSpec((128,), lambda m: (0,)),
                  pl.BlockSpec((128, 64), lambda m: (0, 0)),
                  pl.BlockSpec((64,), lambda m: (0,)),
                  pl.BlockSpec((64, 1), lambda m: (0, 0)),
                  pl.BlockSpec((1,), lambda m: (0,))],
        out_specs=pl.BlockSpec((BM, 1), lambda m: (m, 0)),
        out_shape=jax.ShapeDtypeStruct((EP2, 1), _F32),
    )(h1, W2, b2, W3, b3, W4, b4)
    return jnp.squeeze(o, axis=1)


def _edge_logits_groups(bl, br, nf, attv, outv, maxv, tbuf, B):
    lane17 = lax.iota(_I32, LANES) * 17

    def group(g, _):
        for j in range(LANES):
            i = g * LANES + j
            acc = jnp.zeros((LANES,), _F32)
            for fc in range(nf // LANES):
                sl = pl.ds(fc * LANES, LANES)
                u = bl[i, sl] + br[i, sl]
                t = jnp.maximum(u, 0.2 * u)
                acc = acc + t * attv[pl.ds(fc * LANES, LANES)]
            tbuf[pl.ds(j * 17, LANES)] = acc
        gacc = plsc.load_gather(tbuf, [lane17])
        for k in range(1, LANES):
            gacc = gacc + plsc.load_gather(tbuf, [lane17 + k])
        outv[pl.ds(g * LANES, LANES)] = gacc
        maxv[...] = jnp.maximum(maxv[...], gacc)
        return 0

    lax.fori_loop(0, B // LANES, group, 0)


def _s1_split_body(xlh, xrh, atth, srch, dsth, lgh, wmh,
                   srcv, dstv, bl, br, attv, outv, maxv, tbuf, sem):
    B = 80
    K = 2
    EW = E // NW
    wid = lax.axis_index("s") * NC + lax.axis_index("c")
    base = wid * EW
    pltpu.sync_copy(atth, attv)
    maxv[...] = jnp.full((LANES,), -1e30, _F32)

    def fire(b, off):
        pltpu.sync_copy(srch.at[pl.ds(off, B)], srcv.at[b])
        pltpu.sync_copy(dsth.at[pl.ds(off, B)], dstv.at[b])
        return (pltpu.async_copy(xlh.at[srcv.at[b]], bl.at[b], sem.at[b]),
                pltpu.async_copy(xrh.at[dstv.at[b]], br.at[b], sem.at[b]))

    def drain(b, off, ds):
        ds[0].wait(); ds[1].wait()
        _edge_logits_groups(bl.at[b], br.at[b], 256, attv, outv,
                            maxv, tbuf, B)
        pltpu.sync_copy(outv, lgh.at[pl.ds(off, B)])

    def sup(go, carry):
        off = base + go * (K * B)
        dss = [fire(b2, off + b2 * B) for b2 in range(K)]
        for b2 in range(K):
            drain(b2, off + b2 * B, dss[b2])
        return carry

    nb = EW // B
    lax.fori_loop(0, nb // K, sup, 0)
    off = base + (nb - 1) * B
    drain(0, off, fire(0, off))
    pltpu.sync_copy(maxv, wmh.at[wid])


def _s1_split(xl, xr, att1, src, dst):
    B = 80
    K = 2
    return pl.kernel(
        _s1_split_body,
        out_type=[jax.ShapeDtypeStruct((E,), _F32),
                  jax.ShapeDtypeStruct((NW, LANES), _F32)],
        mesh=_MESH,
        compiler_params=pltpu.CompilerParams(needs_layout_passes=False),
        scratch_types=[pltpu.VMEM((K, B), _I32), pltpu.VMEM((K, B), _I32),
                       pltpu.VMEM((K, B, 256), _F32),
                       pltpu.VMEM((K, B, 256), _F32),
                       pltpu.VMEM((256,), _F32), pltpu.VMEM((B,), _F32),
                       pltpu.VMEM((LANES,), _F32),
                       pltpu.VMEM((17 * LANES,), _F32),
                       pltpu.SemaphoreType.DMA((K,))],
    )(xl, xr, att1, src, dst)


def _s1_full_body(xlh, xrh, atth, srch, dsth, lgh, wmh,
                  srcv, dstv, bl, br, attv, outv, maxv, tbuf, sem):
    B = 80
    K = 5
    EW = E // NW
    wid = lax.axis_index("s") * NC + lax.axis_index("c")
    base = wid * EW
    pltpu.sync_copy(atth, attv)
    maxv[...] = jnp.full((LANES,), -1e30, _F32)

    def sup(go, carry):
        off = base + go * (K * B)
        ds = []
        for b in range(K):
            pltpu.sync_copy(srch.at[pl.ds(off + b * B, B)], srcv.at[b])
            pltpu.sync_copy(dsth.at[pl.ds(off + b * B, B)], dstv.at[b])
            d1 = pltpu.async_copy(xlh.at[srcv.at[b]], bl.at[b], sem.at[b])
            d2 = pltpu.async_copy(xrh.at[dstv.at[b]], br.at[b], sem.at[b])
            ds.append((d1, d2))
        for b in range(K):
            d1, d2 = ds[b]
            d1.wait(); d2.wait()
            _edge_logits_groups(bl.at[b], br.at[b], 128, attv, outv,
                                maxv, tbuf, B)
            pltpu.sync_copy(outv, lgh.at[pl.ds(off + b * B, B)])
        return carry

    lax.fori_loop(0, EW // (K * B), sup, 0)
    pltpu.sync_copy(maxv, wmh.at[wid])


def _s1_full(xl2, xr2, att2, src, dst):
    B = 80
    K = 5
    return pl.kernel(
        _s1_full_body,
        out_type=[jax.ShapeDtypeStruct((E,), _F32),
                  jax.ShapeDtypeStruct((NW, LANES), _F32)],
        mesh=_MESH,
        compiler_params=pltpu.CompilerParams(needs_layout_passes=False),
        scratch_types=[pltpu.VMEM((K, B), _I32), pltpu.VMEM((K, B), _I32),
                       pltpu.VMEM((K, B, 128), _F32),
                       pltpu.VMEM((K, B, 128), _F32),
                       pltpu.VMEM((128,), _F32), pltpu.VMEM((B,), _F32),
                       pltpu.VMEM((LANES,), _F32),
                       pltpu.VMEM((17 * LANES,), _F32),
                       pltpu.SemaphoreType.DMA((K,))],
    )(xl2, xr2, att2, src, dst)


def _gmax(wmv):
    m = wmv[0]

    def mx(k, mm):
        return jnp.maximum(mm, wmv[k])

    m = lax.fori_loop(1, NW, mx, m)
    g = m[0]
    for k in range(1, LANES):
        g = jnp.maximum(g, m[k])
    return g


def _s2_body(lgh, dsth, wmh, dph, lgv, dstv, wmv, denl, sbuf, resv, shared):
    B2 = 2000
    EW = E // NW
    c = lax.axis_index("c")
    s = lax.axis_index("s")
    wid = s * NC + c
    base = wid * EW

    def zz(k, _):
        denl[pl.ds(k * LANES, LANES)] = jnp.zeros((LANES,), _F32)
        return 0

    lax.fori_loop(0, (DH * DW) // LANES, zz, 0)

    pltpu.sync_copy(wmh, wmv)
    g = _gmax(wmv)

    def batch(b, _):
        off = base + b * B2
        pltpu.sync_copy(lgh.at[pl.ds(off, B2)], lgv)
        pltpu.sync_copy(dsth.at[pl.ds(off, B2)], dstv)

        def grp(k, _2):
            sl = pl.ds(k * LANES, LANES)
            ex = jnp.exp(lgv[sl] - g)
            d = dstv[sl]
            plsc.addupdate_scatter(denl, [d], ex)
            return 0

        lax.fori_loop(0, B2 // LANES, grp, 0)
        return 0

    lax.fori_loop(0, EW // B2, batch, 0)

    for t in range(NS):
        pltpu.sync_copy(denl.at[pl.ds(t * DW, DW)], shared.at[t, s])
    plsc.subcore_barrier()
    pltpu.sync_copy(shared.at[s], sbuf)

    def col(k, _):
        sl = pl.ds(k * LANES, LANES)
        a = sbuf[0, sl]
        for p in range(1, NS):
            a = a + sbuf[p, sl]
        resv[sl] = a
        return 0

    lax.fori_loop(0, DW // LANES, col, 0)
    pltpu.sync_copy(resv, dph.at[c, pl.ds(s * DW, DW)])


def _s2(lg, dst, wm):
    return pl.kernel(
        _s2_body,
        out_type=jax.ShapeDtypeStruct((NC, DH * DW), _F32),
        mesh=_MESH,
        compiler_params=pltpu.CompilerParams(needs_layout_passes=False),
        scratch_types=[pltpu.VMEM((2000,), _F32), pltpu.VMEM((2000,), _I32),
                       pltpu.VMEM((NW, LANES), _F32),
                       pltpu.VMEM((DH * DW,), _F32),
                       pltpu.VMEM((NS, DW), _F32), pltpu.VMEM((DW,), _F32),
                       pltpu.VMEM_SHARED((NS, NS, DW), _F32)],
    )(lg, dst, wm)


def _denom_combine(dph, denv, dtmp):
    pltpu.sync_copy(dph.at[0], denv)
    for t in range(DH):
        pltpu.sync_copy(dph.at[1, pl.ds(t * DW, DW)], dtmp)

        def addp(k, _):
            sl = pl.ds(t * DW + k * LANES, LANES)
            denv[sl] = denv[sl] + dtmp[pl.ds(k * LANES, LANES)]
            return 0

        lax.fori_loop(0, DW // LANES, addp, 0)


def _zero_acc(accS, dbuf, s):
    def zd(k, _):
        dbuf[k >> 3, pl.ds((k & 7) * LANES, LANES)] = jnp.zeros((LANES,), _F32)
        return 0

    lax.fori_loop(0, 40 * 8, zd, 0)
    for j in range(CHK // 40):
        pltpu.sync_copy(dbuf, accS.at[pl.ds(s * CHK + j * 40, 40)])


def _alpha_batch(lgv, dstv, denv, alv, g, B):
    for gi in range(B // LANES):
        sl = pl.ds(gi * LANES, LANES)
        ex = jnp.exp(lgv[sl] - g)
        d = dstv[sl]
        den = plsc.load_gather(denv, [d])
        alv[sl] = ex / (den + 1e-16)


def _scale_rows(rows, alv, B):
    def sc(gi, _):
        av = alv[pl.ds(gi * LANES, LANES)]
        for j in range(LANES):
            i = gi * LANES + j
            a = av[j]
            for kk in range(8):
                sl = pl.ds(kk * LANES, LANES)
                rows[i, sl] = rows[i, sl] * a
        return 0

    lax.fori_loop(0, B // LANES, sc, 0)


def _s3_split_body(srch, dsth, lgh, wmh, dph, xstk, bch, outa, outb,
                   srcv, dstv, lgv, alv, rows, denv, dtmp, wmv, bcv, dbuf,
                   accS, sem):
    B = 80
    K = 2
    EC = E // NS
    c = lax.axis_index("c")
    s = lax.axis_index("s")
    base = s * EC
    coff = c * N

    pltpu.sync_copy(wmh, wmv)
    g = _gmax(wmv)
    _denom_combine(dph, denv, dtmp)
    pltpu.sync_copy(bch.at[c], bcv)
    _zero_acc(accS, dbuf, s)
    plsc.subcore_barrier()

    def fire(b, off):
        pltpu.sync_copy(srch.at[pl.ds(off, B)], srcv.at[b])
        pltpu.sync_copy(dsth.at[pl.ds(off, B)], dstv.at[b])
        pltpu.sync_copy(lgh.at[pl.ds(off, B)], lgv.at[b])
        sv = srcv.at[b]

        def addo(gi, _):
            sl = pl.ds(gi * LANES, LANES)
            sv[sl] = sv[sl] + coff
            return 0

        lax.fori_loop(0, B // LANES, addo, 0)
        return pltpu.async_copy(xstk.at[srcv.at[b]], rows.at[b], sem.at[b])

    def compute(b, d):
        d.wait()
        _alpha_batch(lgv.at[b], dstv.at[b], denv, alv, g, B)
        _scale_rows(rows.at[b], alv, B)
        return pltpu.async_copy(rows.at[b], accS.at[dstv.at[b]], sem.at[b],
                                add=True)

    def sup(go, carry):
        off = base + go * (K * B)
        dss = [fire(b2, off + b2 * B) for b2 in range(K)]
        scs = [compute(b2, dss[b2]) for b2 in range(K)]
        for sc2 in scs:
            sc2.wait()
        return carry

    lax.fori_loop(0, EC // (K * B), sup, 0)
    plsc.subcore_barrier()

    for j in range(CHK // 40):
        r0 = s * CHK + j * 40
        pltpu.sync_copy(accS.at[pl.ds(r0, 40)], dbuf)

        def br(i, _):
            for kk in range(8):
                sl = pl.ds(kk * LANES, LANES)
                dbuf[i, sl] = jnp.maximum(dbuf[i, sl] + bcv[sl], 0.0)
            return 0

        lax.fori_loop(0, 40, br, 0)

        @pl.when(c == 0)
        def _():
            pltpu.sync_copy(dbuf, outa.at[pl.ds(r0, 40)])

        @pl.when(c == 1)
        def _():
            pltpu.sync_copy(dbuf, outb.at[pl.ds(r0, 40)])


def _s3_split(src, dst, lg, wm, dp, xstk, bc1r):
    B = 80
    K = 2
    return pl.kernel(
        _s3_split_body,
        out_type=[jax.ShapeDtypeStruct((NPS, 128), _F32)] * 2,
        mesh=_MESH,
        compiler_params=pltpu.CompilerParams(needs_layout_passes=False),
        scratch_types=[pltpu.VMEM((K, B), _I32), pltpu.VMEM((K, B), _I32),
                       pltpu.VMEM((K, B), _F32), pltpu.VMEM((B,), _F32),
                       pltpu.VMEM((K, B, 128), _F32),
                       pltpu.VMEM((DH * DW,), _F32), pltpu.VMEM((DW,), _F32),
                       pltpu.VMEM((NW, LANES), _F32), pltpu.VMEM((128,), _F32),
                       pltpu.VMEM((40, 128), _F32),
                       pltpu.VMEM_SHARED((NPS, 128), _F32),
                       pltpu.SemaphoreType.DMA((K,))],
    )(src, dst, lg, wm, dp, xstk, bc1r)


def _s3_full_body(srch, dsth, lgh, wmh, dph, xlh, p0h, p1h,
                  srcv, dstv, lgv, alv, rows, denv, dtmp, wmv, dbuf,
                  accS, sem):
    B = 80
    K = 2
    EC = E // NW
    c = lax.axis_index("c")
    s = lax.axis_index("s")
    base = c * (E // NC) + s * EC

    pltpu.sync_copy(wmh, wmv)
    g = _gmax(wmv)
    _denom_combine(dph, denv, dtmp)
    _zero_acc(accS, dbuf, s)
    plsc.subcore_barrier()

    def fire(b, off):
        pltpu.sync_copy(srch.at[pl.ds(off, B)], srcv.at[b])
        pltpu.sync_copy(dsth.at[pl.ds(off, B)], dstv.at[b])
        pltpu.sync_copy(lgh.at[pl.ds(off, B)], lgv.at[b])
        return pltpu.async_copy(xlh.at[srcv.at[b]], rows.at[b], sem.at[b])

    def compute(b, d):
        d.wait()
        _alpha_batch(lgv.at[b], dstv.at[b], denv, alv, g, B)
        _scale_rows(rows.at[b], alv, B)
        return pltpu.async_copy(rows.at[b], accS.at[dstv.at[b]], sem.at[b],
                                add=True)

    def sup(go, carry):
        off = base + go * (K * B)
        dss = [fire(b2, off + b2 * B) for b2 in range(K)]
        scs = [compute(b2, dss[b2]) for b2 in range(K)]
        for sc2 in scs:
            sc2.wait()
        return carry

    nb = EC // B
    lax.fori_loop(0, nb // K, sup, 0)
    off = base + (nb - 1) * B
    compute(0, fire(0, off)).wait()
    plsc.subcore_barrier()

    for j in range(CHK // 40):
        r0 = s * CHK + j * 40
        pltpu.sync_copy(accS.at[pl.ds(r0, 40)], dbuf)

        @pl.when(c == 0)
        def _():
            pltpu.sync_copy(dbuf, p0h.at[pl.ds(r0, 40)])

        @pl.when(c == 1)
        def _():
            pltpu.sync_copy(dbuf, p1h.at[pl.ds(r0, 40)])


def _s3_full(src, dst, lg, wm, dp, xl2):
    B = 80
    K = 2
    return pl.kernel(
        _s3_full_body,
        out_type=[jax.ShapeDtypeStruct((NPS, 128), _F32)] * 2,
        mesh=_MESH,
        compiler_params=pltpu.CompilerParams(needs_layout_passes=False),
        scratch_types=[pltpu.VMEM((K, B), _I32), pltpu.VMEM((K, B), _I32),
                       pltpu.VMEM((K, B), _F32), pltpu.VMEM((B,), _F32),
                       pltpu.VMEM((K, B, 128), _F32),
                       pltpu.VMEM((DH * DW,), _F32), pltpu.VMEM((DW,), _F32),
                       pltpu.VMEM((NW, LANES), _F32),
                       pltpu.VMEM((40, 128), _F32),
                       pltpu.VMEM_SHARED((NPS, 128), _F32),
                       pltpu.SemaphoreType.DMA((K,))],
    )(src, dst, lg, wm, dp, xl2)


def _s4_body(zuh, zvh, e0h, e1h, b1h, h1h, uv, vv, bufu, bufv, b1v, sem):
    B = 128
    EW = EP2 // NW
    wid = lax.axis_index("s") * NC + lax.axis_index("c")
    base = wid * EW
    pltpu.sync_copy(b1h, b1v)

    def batch(b, _):
        off = base + b * B
        pltpu.sync_copy(e0h.at[pl.ds(off, B)], uv)
        pltpu.sync_copy(e1h.at[pl.ds(off, B)], vv)
        d1 = pltpu.async_copy(zuh.at[uv], bufu, sem)
        d2 = pltpu.async_copy(zvh.at[vv], bufv, sem)
        d1.wait(); d2.wait()

        def row(i, _2):
            for kk in range(8):
                sl = pl.ds(kk * LANES, LANES)
                bufu[i, sl] = jnp.maximum(bufu[i, sl] + bufv[i, sl] + b1v[sl], 0.0)
            return 0

        lax.fori_loop(0, B, row, 0)
        pltpu.sync_copy(bufu, h1h.at[pl.ds(off, B)])
        return 0

    lax.fori_loop(0, EW // B, batch, 0)


def _s4(ZU, ZV, ei0, ei1, b1):
    B = 128
    return pl.kernel(
        _s4_body,
        out_type=jax.ShapeDtypeStruct((EP2, 128), _F32),
        mesh=_MESH,
        compiler_params=pltpu.CompilerParams(needs_layout_passes=False),
        scratch_types=[pltpu.VMEM((B,), _I32), pltpu.VMEM((B,), _I32),
                       pltpu.VMEM((B, 128), _F32), pltpu.VMEM((B, 128), _F32),
                       pltpu.VMEM((128,), _F32), pltpu.SemaphoreType.DMA],
    )(ZU, ZV, ei0, ei1, b1)


def kernel(x, edge_index, pos_edge_index, neg_edge_index,
           Wl1, Wr1, att1, bc1, Wl2, Wr2, att2, bc2,
           W1, b1, W2, b2, W3, b3, W4, b4):
    src = edge_index[0]
    dst = edge_index[1]

    xl1, xr1, xstk = _m1(x, Wl1, Wr1)
    lg1, wm1 = _s1_split(xl1, xr1, att1, src, dst)
    dp1 = _s2(lg1, dst, wm1)
    z1a, z1b = _s3_split(src, dst, lg1, wm1, dp1, xstk,
                         bc1.reshape(2, 128))

    xl2, xr2 = _m2(z1a[:N], z1b[:N], Wl2, Wr2)
    lg2, wm2 = _s1_full(xl2, xr2, att2, src, dst)
    dp2 = _s2(lg2, dst, wm2)
    p0, p1 = _s3_full(src, dst, lg2, wm2, dp2, xl2)

    ZU, ZV = _m3(p0[:N], p1[:N], bc2, W1)
    ei0 = jnp.concatenate([pos_edge_index[0], neg_edge_index[0]])
    ei1 = jnp.concatenate([pos_edge_index[1], neg_edge_index[1]])
    h1 = _s4(ZU, ZV, ei0, ei1, b1)
    return _m4(h1, W2, b2, W3, b3, W4, b4)
